# Initial kernel scaffold; baseline (speedup 1.0000x reference)
import jax
import jax.numpy as jnp
from jax.experimental import pallas as pl


def kernel(x):
    raise NotImplementedError("write your pallas kernel here")


# speedup vs baseline: 3.3994x; 3.3994x over previous
import functools

import jax
import jax.numpy as jnp
from jax import lax
from jax.experimental import pallas as pl
from jax.experimental.pallas import tpu as pltpu
from jax.experimental.pallas import tpu_sc as plsc

N_ROWS = 64
N_COLS = 8192
KTH = 256
NUM_CORES = 2
NUM_SUBCORES = 16
NW = NUM_CORES * NUM_SUBCORES
ROWS_PER_W = N_ROWS // NW
L = 16
NCHUNK = N_COLS // L
TOP_I = -(2 ** 31)


def _popcnt(mask):
    return plsc.all_reduce_population_count(mask)[0]


def _sc_kthvalue(x_bits):
    mesh = plsc.VectorSubcoreMesh(
        core_axis_name="c", subcore_axis_name="s",
        num_cores=NUM_CORES, num_subcores=NUM_SUBCORES)

    @functools.partial(
        pl.kernel,
        out_type=(jax.ShapeDtypeStruct((NW, L), jnp.int32),
                  jax.ShapeDtypeStruct((NW, L), jnp.int32)),
        mesh=mesh,
        compiler_params=pltpu.CompilerParams(needs_layout_passes=False),
        scratch_types=[
            pltpu.VMEM((N_COLS,), jnp.int32),
            pltpu.VMEM((N_COLS,), jnp.int32),
            pltpu.VMEM((N_COLS,), jnp.int32),
            pltpu.VMEM((L,), jnp.int32),
            pltpu.VMEM((L,), jnp.int32),
        ],
    )
    def body(x_hbm, vout_hbm, iout_hbm, kbuf0, kbuf1, ibuf, vstage, istage):
        wid = lax.axis_index("s") * NUM_CORES + lax.axis_index("c")
        io = lax.iota(jnp.int32, L)

        kbufs = (kbuf0, kbuf1)
        for row in range(ROWS_PER_W):
            pltpu.sync_copy(x_hbm.at[wid * ROWS_PER_W + row], kbufs[row])

        res_v = jnp.zeros((L,), jnp.int32)
        res_i = jnp.zeros((L,), jnp.int32)

        for row in range(ROWS_PER_W):
            krow = kbufs[row]

            def pass0(j, cvec):
                base = j * L
                bits = krow[pl.ds(base, L)]
                top = jnp.int32(TOP_I)
                bits = jnp.where(bits == top, jnp.int32(0), bits)
                m = lax.shift_right_arithmetic(bits, 31)
                key = lax.bitwise_xor(bits, lax.bitwise_or(m, top))
                krow[pl.ds(base, L)] = key
                ibuf[pl.ds(base, L)] = io + base
                return cvec + _popcnt(key >= 0)

            c0 = lax.fori_loop(0, NCHUNK, pass0, jnp.int32(0))

            r0 = jnp.int32(KTH)
            go_low = r0 <= c0
            sel0 = jnp.where(go_low, jnp.int32(0), jnp.int32(1))
            r0 = jnp.where(go_low, r0, r0 - c0)
            n_after0 = jnp.where(go_low, c0, jnp.int32(N_COLS) - c0)

            def level_cond(st):
                pb, _, _, _, _ = st
                return pb >= 0

            def level_body(st):
                pb, sel, n, n_after, r = st
                cb = jnp.maximum(pb - 1, jnp.int32(0))
                nchnk = lax.div(n + jnp.int32(L - 1), jnp.int32(L))

                def chunk(j, carry):
                    off, cv = carry
                    base = j * L
                    kv = krow[pl.ds(base, L)]
                    iv = ibuf[pl.ds(base, L)]
                    lanes_ok = (io + base) < n
                    bitv = lax.bitwise_and(
                        lax.shift_right_logical(kv, pb), jnp.int32(1))
                    keep = jnp.logical_and(bitv == sel, lanes_ok)
                    plsc.store_compressed(krow.at[pl.ds(off, L)], kv,
                                          mask=keep)
                    plsc.store_compressed(ibuf.at[pl.ds(off, L)], iv,
                                          mask=keep)
                    nb = lax.bitwise_and(
                        lax.shift_right_logical(kv, cb), jnp.int32(1))
                    nxt0 = jnp.logical_and(keep, nb == 0)
                    cv = cv + _popcnt(nxt0)
                    return off + _popcnt(keep), cv

                off, c = lax.fori_loop(
                    0, nchnk, chunk, (jnp.int32(0), jnp.int32(0)))
                n_new = n_after
                done = jnp.logical_or(pb == 0, n_new <= 1)
                glow = r <= c
                sel_n = jnp.where(glow, jnp.int32(0), jnp.int32(1))
                r_n = jnp.where(done, r, jnp.where(glow, r, r - c))
                n_after_n = jnp.where(glow, c, n_new - c)
                pb_n = jnp.where(done, jnp.int32(-1), pb - 1)
                return (pb_n, sel_n, n_new, n_after_n, r_n)

            st = (jnp.int32(31), sel0, jnp.int32(N_COLS), n_after0, r0)
            _, _, _, _, r_fin = lax.while_loop(level_cond, level_body, st)

            pos = jnp.full((L,), r_fin - 1, jnp.int32)
            kv_ans = plsc.load_gather(krow, [pos])
            iv_ans = plsc.load_gather(ibuf, [pos])
            lane = io == row
            res_v = jnp.where(lane, kv_ans, res_v)
            res_i = jnp.where(lane, iv_ans, res_i)

        inv = jnp.where(res_v < 0,
                        lax.bitwise_xor(res_v, jnp.int32(TOP_I)),
                        lax.bitwise_xor(res_v, jnp.int32(-1)))
        vstage[...] = inv
        istage[...] = res_i
        pltpu.sync_copy(vstage, vout_hbm.at[wid])
        pltpu.sync_copy(istage, iout_hbm.at[wid])

    return body(x_bits)


def kernel(x):
    xb = lax.bitcast_convert_type(x, jnp.int32)
    vbits, inds = _sc_kthvalue(xb)
    values = lax.bitcast_convert_type(
        vbits[:, :ROWS_PER_W].reshape(N_ROWS), jnp.float32)
    indices = inds[:, :ROWS_PER_W].reshape(N_ROWS)
    return values, indices.astype(jnp.int64)


# speedup vs baseline: 3.5579x; 1.0466x over previous
import functools

import jax
import jax.numpy as jnp
from jax import lax
from jax.experimental import pallas as pl
from jax.experimental.pallas import tpu as pltpu
from jax.experimental.pallas import tpu_sc as plsc

N_ROWS = 64
N_COLS = 8192
KTH = 256
NUM_CORES = 2
NUM_SUBCORES = 16
NW = NUM_CORES * NUM_SUBCORES
ROWS_PER_W = N_ROWS // NW
L = 16
U = 4
UL = U * L
TOP_I = -(2 ** 31)


def _popcnt(mask):
    return plsc.all_reduce_population_count(mask)[0]


def _sc_kthvalue(x_bits):
    mesh = plsc.VectorSubcoreMesh(
        core_axis_name="c", subcore_axis_name="s",
        num_cores=NUM_CORES, num_subcores=NUM_SUBCORES)

    @functools.partial(
        pl.kernel,
        out_type=(jax.ShapeDtypeStruct((NW, L), jnp.int32),
                  jax.ShapeDtypeStruct((NW, L), jnp.int32)),
        mesh=mesh,
        compiler_params=pltpu.CompilerParams(needs_layout_passes=False),
        scratch_types=[
            pltpu.VMEM((N_COLS,), jnp.int32),
            pltpu.VMEM((N_COLS,), jnp.int32),
            pltpu.VMEM((N_COLS,), jnp.int32),
            pltpu.VMEM((L,), jnp.int32),
            pltpu.VMEM((L,), jnp.int32),
            pltpu.VMEM((L,), jnp.int32),
        ],
    )
    def body(x_hbm, vout_hbm, iout_hbm, kbuf0, kbuf1, ibuf, bfly, vstage,
             istage):
        wid = lax.axis_index("s") * NUM_CORES + lax.axis_index("c")
        io = lax.iota(jnp.int32, L)
        perms = tuple(lax.bitwise_xor(io, jnp.int32(1 << p))
                      for p in range(3, -1, -1))
        one = jnp.int32(1)
        zero = jnp.int32(0)
        top = jnp.int32(TOP_I)

        def lane_sum(v):
            for p in perms:
                bfly[...] = v
                v = v + plsc.load_gather(bfly, [p])
            return v[0]

        kbufs = (kbuf0, kbuf1)
        for row in range(ROWS_PER_W):
            pltpu.sync_copy(x_hbm.at[wid * ROWS_PER_W + row], kbufs[row])

        res_v = jnp.zeros((L,), jnp.int32)
        res_i = jnp.zeros((L,), jnp.int32)

        for row in range(ROWS_PER_W):
            krow = kbufs[row]

            def pass_a(j, cv):
                base = j * UL
                for u in range(U):
                    b = krow[pl.ds(base + u * L, L)]
                    b = jnp.where(b == top, zero, b)
                    cv = cv + jnp.where(b < 0, one, zero)
                return cv

            c0 = lane_sum(lax.fori_loop(0, N_COLS // UL, pass_a,
                                        jnp.zeros((L,), jnp.int32)))

            r0 = jnp.int32(KTH)
            glow0 = r0 <= c0
            selb0 = jnp.logical_not(glow0)
            r0 = jnp.where(glow0, r0, r0 - c0)
            n_after0 = jnp.where(glow0, c0, jnp.int32(N_COLS) - c0)

            def pass_b(j, carry):
                off, cv = carry
                base = j * UL
                for u in range(U):
                    bs = base + u * L
                    b = krow[pl.ds(bs, L)]
                    b = jnp.where(b == top, zero, b)
                    m = lax.shift_right_arithmetic(b, 31)
                    key = lax.bitwise_xor(b, lax.bitwise_or(m, top))
                    keep = (key >= 0) != selb0
                    plsc.store_compressed(krow.at[pl.ds(off, L)], key,
                                          mask=keep)
                    plsc.store_compressed(ibuf.at[pl.ds(off, L)], io + bs,
                                          mask=keep)
                    b30 = lax.bitwise_and(lax.shift_right_logical(key, 30),
                                          one)
                    nxt0 = jnp.logical_and(keep, b30 == 0)
                    cv = cv + jnp.where(nxt0, one, zero)
                    off = off + _popcnt(keep)
                return off, cv

            _, cvb = lax.fori_loop(0, N_COLS // UL, pass_b,
                                   (zero, jnp.zeros((L,), jnp.int32)))
            c1 = lane_sum(cvb)

            glow = r0 <= c1
            sel1 = jnp.where(glow, zero, one)
            r1a = jnp.where(glow, r0, r0 - c1)
            n_after1 = jnp.where(glow, c1, n_after0 - c1)
            done1 = n_after0 <= 1
            pb1 = jnp.where(done1, jnp.int32(-1), jnp.int32(30))
            r1 = jnp.where(done1, r0, r1a)

            def level_cond(st):
                pb, _, _, _, _ = st
                return pb >= 0

            def level_body(st):
                pb, sel, n, n_after, r = st
                cb = jnp.maximum(pb - 1, zero)
                nit = lax.div(n + jnp.int32(UL - 1), jnp.int32(UL))

                def chunk(j, carry):
                    off, cv = carry
                    base = j * UL
                    for u in range(U):
                        bs = base + u * L
                        kv = krow[pl.ds(bs, L)]
                        iv = ibuf[pl.ds(bs, L)]
                        ok = (io + bs) < n
                        bitv = lax.bitwise_and(
                            lax.shift_right_logical(kv, pb), one)
                        keep = jnp.logical_and(bitv == sel, ok)
                        plsc.store_compressed(krow.at[pl.ds(off, L)], kv,
                                              mask=keep)
                        plsc.store_compressed(ibuf.at[pl.ds(off, L)], iv,
                                              mask=keep)
                        nb = lax.bitwise_and(
                            lax.shift_right_logical(kv, cb), one)
                        nxt0 = jnp.logical_and(keep, nb == 0)
                        cv = cv + jnp.where(nxt0, one, zero)
                        off = off + _popcnt(keep)
                    return off, cv

                _, cv = lax.fori_loop(0, nit, chunk,
                                      (zero, jnp.zeros((L,), jnp.int32)))
                c = lane_sum(cv)
                n_new = n_after
                done = jnp.logical_or(pb == 0, n_new <= 1)
                gl = r <= c
                sel_n = jnp.where(gl, zero, one)
                r_n = jnp.where(done, r, jnp.where(gl, r, r - c))
                n_after_n = jnp.where(gl, c, n_new - c)
                pb_n = jnp.where(done, jnp.int32(-1), pb - 1)
                return (pb_n, sel_n, n_new, n_after_n, r_n)

            st = (pb1, sel1, n_after0, n_after1, r1)
            _, _, _, _, r_fin = lax.while_loop(level_cond, level_body, st)

            pos = jnp.full((L,), r_fin - 1, jnp.int32)
            kv_ans = plsc.load_gather(krow, [pos])
            iv_ans = plsc.load_gather(ibuf, [pos])
            lane = io == row
            res_v = jnp.where(lane, kv_ans, res_v)
            res_i = jnp.where(lane, iv_ans, res_i)

        inv = jnp.where(res_v < 0,
                        lax.bitwise_xor(res_v, top),
                        lax.bitwise_xor(res_v, jnp.int32(-1)))
        vstage[...] = inv
        istage[...] = res_i
        pltpu.sync_copy(vstage, vout_hbm.at[wid])
        pltpu.sync_copy(istage, iout_hbm.at[wid])

    return body(x_bits)


def kernel(x):
    xb = lax.bitcast_convert_type(x, jnp.int32)
    vbits, inds = _sc_kthvalue(xb)
    values = lax.bitcast_convert_type(
        vbits[:, :ROWS_PER_W].reshape(N_ROWS), jnp.float32)
    indices = inds[:, :ROWS_PER_W].reshape(N_ROWS)
    return values, indices.astype(jnp.int64)
